# baseline (speedup 1.0000x reference)
import functools

import jax
import jax.numpy as jnp
from jax import lax
from jax.experimental import pallas as pl
from jax.experimental.pallas import tpu as pltpu
from jax.experimental.pallas import tpu_sc as plsc

_B = 16384
_D = 4096
_NC = 2
_NS = 16
_NW = _NC * _NS
_RPW = _B // _NW
_C = 8
_LANES = 16


def _body(x_hbm, idx_hbm, m_hbm, out_hbm, idx_v, x_v, g_v, m_v, sem):
    wid = lax.axis_index("s") * _NC + lax.axis_index("c")
    base = wid * _RPW
    pltpu.sync_copy(idx_hbm.at[pl.ds(base, _RPW)], idx_v)

    lane = lax.iota(jnp.int32, _LANES)
    byte_in_word = lane & 3
    bit_sel = jnp.int32(1) << (byte_in_word << 3)

    def chunk(ci, _):
        row0 = base + ci * _C
        gather = pltpu.async_copy(x_hbm.at[idx_v.at[pl.ds(ci * _C, _C)]],
                                  g_v, sem)
        pltpu.sync_copy(x_hbm.at[pl.ds(row0, _C)], x_v)
        pltpu.sync_copy(m_hbm.at[pl.ds(row0, _C)], m_v)
        gather.wait()

        def group(e, _):
            for r in range(_C):
                mw = m_v[r, pl.ds(e * 16, 16)]
                for k in range(4):
                    widx = (lane >> 2) + (4 * k)
                    sel = mw.at[widx].get(mode="promise_in_bounds")
                    keep = (sel & bit_sel) != 0
                    off = e * 64 + k * 16
                    xv = x_v[r, pl.ds(off, _LANES)]
                    gv = g_v[r, pl.ds(off, _LANES)]
                    x_v[r, pl.ds(off, _LANES)] = jnp.where(keep, gv, xv)
            return ()

        lax.fori_loop(0, _D // 64, group, (), unroll=False)
        pltpu.sync_copy(x_v, out_hbm.at[pl.ds(row0, _C)])
        return ()

    lax.fori_loop(0, _RPW // _C, chunk, (), unroll=False)


@jax.jit
def _cutmix_sc(x, idx, m8):
    mesh = plsc.VectorSubcoreMesh(core_axis_name="c", subcore_axis_name="s",
                                  num_cores=_NC, num_subcores=_NS)
    run = pl.kernel(
        _body,
        out_type=jax.ShapeDtypeStruct((_B, _D), jnp.float32),
        mesh=mesh,
        scratch_types=[
            pltpu.VMEM((_RPW,), jnp.int32),
            pltpu.VMEM((_C, _D), jnp.float32),
            pltpu.VMEM((_C, _D), jnp.float32),
            pltpu.VMEM((_C, _D // 4), jnp.int32),
            pltpu.SemaphoreType.DMA,
        ],
    )
    return run(x, idx, m8)


def kernel(x, shuffled_idx, mask):
    m32 = lax.bitcast_convert_type(
        mask.view(jnp.int8).reshape(_B, _D // 4, 4), jnp.int32)
    return _cutmix_sc(x, shuffled_idx, m32)


# speedup vs baseline: 1.2690x; 1.2690x over previous
import functools

import jax
import jax.numpy as jnp
from jax import lax
from jax.experimental import pallas as pl
from jax.experimental.pallas import tpu as pltpu
from jax.experimental.pallas import tpu_sc as plsc

_B = 16384
_D = 4096
_NC = 2
_NS = 16
_NW = _NC * _NS
_RPW = _B // _NW
_C = 8
_W = 2048
_NH = _D // _W
_NCH = (_RPW // _C) * _NH
_LANES = 16


def _body(x_hbm, idx_hbm, m_hbm, out_hbm,
          idx_v, x_v, g_v, m_v, o_v, gsem, lsem, osem):
    wid = lax.axis_index("s") * _NC + lax.axis_index("c")
    base = wid * _RPW
    pltpu.sync_copy(idx_hbm.at[pl.ds(base, _RPW)], idx_v)

    lane = lax.iota(jnp.int32, _LANES)
    widx = lane >> 2
    bit_sel = jnp.int32(1) << ((lane & 3) << 3)

    m32 = m_hbm

    def chunk_coords(ci):
        row0 = base + (ci // _NH) * _C
        col0 = pl.multiple_of((ci % _NH) * _W, _W)
        colm0 = pl.multiple_of((ci % _NH) * (_W // 4), _W // 4)
        return row0, col0, colm0

    def issue_in(ci, b):
        row0, col0, colm0 = chunk_coords(ci)
        pltpu.async_copy(
            x_hbm.at[idx_v.at[pl.ds((ci // _NH) * _C, _C)],
                     pl.ds(col0, _W)],
            g_v.at[b], gsem.at[b])
        pltpu.async_copy(x_hbm.at[pl.ds(row0, _C), pl.ds(col0, _W)],
                         x_v.at[b], lsem.at[b])
        pltpu.async_copy(m32.at[pl.ds(row0, _C), pl.ds(colm0, _W // 4)],
                         m_v.at[b], lsem.at[b])

    def wait_in(ci, b):
        row0, col0, colm0 = chunk_coords(ci)
        pltpu.make_async_copy(
            x_hbm.at[idx_v.at[pl.ds((ci // _NH) * _C, _C)],
                     pl.ds(col0, _W)],
            g_v.at[b], gsem.at[b]).wait()
        pltpu.make_async_copy(x_hbm.at[pl.ds(row0, _C), pl.ds(col0, _W)],
                              x_v.at[b], lsem.at[b]).wait()
        pltpu.make_async_copy(m32.at[pl.ds(row0, _C),
                                     pl.ds(colm0, _W // 4)],
                              m_v.at[b], lsem.at[b]).wait()

    def compute(b):
        def group(e, _):
            for r in range(_C):
                mw = m_v[b, r, pl.ds(e * 16, 16)]
                for k in range(4):
                    sel = mw.at[widx + (4 * k)].get(mode="promise_in_bounds")
                    keep = (sel & bit_sel) != 0
                    off = e * 64 + k * 16
                    xv = x_v[b, r, pl.ds(off, _LANES)]
                    gv = g_v[b, r, pl.ds(off, _LANES)]
                    o_v[b, r, pl.ds(off, _LANES)] = jnp.where(keep, gv, xv)
            return ()

        lax.fori_loop(0, _W // 64, group, (), unroll=False)

    def issue_out(ci, b):
        row0, col0, _cm = chunk_coords(ci)
        pltpu.async_copy(o_v.at[b],
                         out_hbm.at[pl.ds(row0, _C), pl.ds(col0, _W)],
                         osem.at[b])

    def wait_out(ci, b):
        row0, col0, _cm = chunk_coords(ci)
        pltpu.make_async_copy(o_v.at[b],
                              out_hbm.at[pl.ds(row0, _C), pl.ds(col0, _W)],
                              osem.at[b]).wait()

    issue_in(0, 0)

    def step(i, _):
        for b in range(2):
            ci = 2 * i + b
            nb = 1 - b

            @pl.when(ci + 1 < _NCH)
            def _():
                issue_in(ci + 1, nb)

            wait_in(ci, b)

            @pl.when(ci >= 2)
            def _():
                wait_out(ci - 2, b)

            compute(b)
            issue_out(ci, b)
        return ()

    lax.fori_loop(0, _NCH // 2, step, (), unroll=False)
    wait_out(_NCH - 2, 0)
    wait_out(_NCH - 1, 1)


@jax.jit
def _cutmix_sc(x, idx, m8):
    mesh = plsc.VectorSubcoreMesh(core_axis_name="c", subcore_axis_name="s",
                                  num_cores=_NC, num_subcores=_NS)
    run = pl.kernel(
        _body,
        out_type=jax.ShapeDtypeStruct((_B, _D), jnp.float32),
        mesh=mesh,
        scratch_types=[
            pltpu.VMEM((_RPW,), jnp.int32),
            pltpu.VMEM((2, _C, _W), jnp.float32),
            pltpu.VMEM((2, _C, _W), jnp.float32),
            pltpu.VMEM((2, _C, _W // 4), jnp.int32),
            pltpu.VMEM((2, _C, _W), jnp.float32),
            pltpu.SemaphoreType.DMA((2,)),
            pltpu.SemaphoreType.DMA((2,)),
            pltpu.SemaphoreType.DMA((2,)),
        ],
    )
    return run(x, idx, m8)


def kernel(x, shuffled_idx, mask):
    m32 = lax.bitcast_convert_type(
        mask.view(jnp.int8).reshape(_B, _D // 4, 4), jnp.int32)
    return _cutmix_sc(x, shuffled_idx, m32)


# speedup vs baseline: 4.6968x; 3.7012x over previous
import functools

import jax
import jax.numpy as jnp
from jax import lax
from jax.experimental import pallas as pl
from jax.experimental.pallas import tpu as pltpu
from jax.experimental.pallas import tpu_sc as plsc

_B = 16384
_D = 4096
_NC = 2
_NS = 16
_NW = _NC * _NS
_RPW = _B // _NW
_C = 8
_W = 2048
_NH = _D // _W
_NCH = (_RPW // _C) * _NH
_LANES = 16


def _body(x_hbm, idx_hbm, m_hbm, out_hbm,
          idx_v, x_v, g_v, m_v, o_v, gsem, lsem, osem):
    wid = lax.axis_index("s") * _NC + lax.axis_index("c")
    base = wid * _RPW
    pltpu.sync_copy(idx_hbm.at[pl.ds(base, _RPW)], idx_v)

    m32 = m_hbm.bitcast(jnp.int32)

    def chunk_coords(ci):
        row0 = base + (ci // _NH) * _C
        col0 = pl.multiple_of((ci % _NH) * _W, _W)
        q0 = pl.multiple_of(row0 >> 2, 2)
        return row0, col0, q0

    def issue_in(ci, b):
        row0, col0, q0 = chunk_coords(ci)
        pltpu.async_copy(
            x_hbm.at[idx_v.at[pl.ds((ci // _NH) * _C, _C)],
                     pl.ds(col0, _W)],
            g_v.at[b], gsem.at[b])
        pltpu.async_copy(x_hbm.at[pl.ds(row0, _C), pl.ds(col0, _W)],
                         x_v.at[b], lsem.at[b])
        pltpu.async_copy(m32.at[pl.ds(q0, _C // 4), pl.ds(col0, _W)],
                         m_v.at[b], lsem.at[b])

    def wait_in(ci, b):
        row0, col0, q0 = chunk_coords(ci)
        pltpu.make_async_copy(
            x_hbm.at[idx_v.at[pl.ds((ci // _NH) * _C, _C)],
                     pl.ds(col0, _W)],
            g_v.at[b], gsem.at[b]).wait()
        pltpu.make_async_copy(x_hbm.at[pl.ds(row0, _C), pl.ds(col0, _W)],
                              x_v.at[b], lsem.at[b]).wait()
        pltpu.make_async_copy(m32.at[pl.ds(q0, _C // 4), pl.ds(col0, _W)],
                              m_v.at[b], lsem.at[b]).wait()

    def compute(b):
        def group(e, _):
            off = e * _LANES
            for q in range(_C // 4):
                mw = m_v[b, q, pl.ds(off, _LANES)]
                for s in range(4):
                    r = 4 * q + s
                    keep = (mw & jnp.int32(1 << (8 * s))) != 0
                    xv = x_v[b, r, pl.ds(off, _LANES)]
                    gv = g_v[b, r, pl.ds(off, _LANES)]
                    o_v[b, r, pl.ds(off, _LANES)] = jnp.where(keep, gv, xv)
            return ()

        lax.fori_loop(0, _W // _LANES, group, (), unroll=False)

    def issue_out(ci, b):
        row0, col0, _q = chunk_coords(ci)
        pltpu.async_copy(o_v.at[b],
                         out_hbm.at[pl.ds(row0, _C), pl.ds(col0, _W)],
                         osem.at[b])

    def wait_out(ci, b):
        row0, col0, _q = chunk_coords(ci)
        pltpu.make_async_copy(o_v.at[b],
                              out_hbm.at[pl.ds(row0, _C), pl.ds(col0, _W)],
                              osem.at[b]).wait()

    issue_in(0, 0)

    def step(i, _):
        for b in range(2):
            ci = 2 * i + b
            nb = 1 - b

            @pl.when(ci + 1 < _NCH)
            def _():
                issue_in(ci + 1, nb)

            wait_in(ci, b)

            @pl.when(ci >= 2)
            def _():
                wait_out(ci - 2, b)

            compute(b)
            issue_out(ci, b)
        return ()

    lax.fori_loop(0, _NCH // 2, step, (), unroll=False)
    wait_out(_NCH - 2, 0)
    wait_out(_NCH - 1, 1)


@jax.jit
def _cutmix_sc(x, idx, m8):
    mesh = plsc.VectorSubcoreMesh(core_axis_name="c", subcore_axis_name="s",
                                  num_cores=_NC, num_subcores=_NS)
    run = pl.kernel(
        _body,
        out_type=jax.ShapeDtypeStruct((_B, _D), jnp.float32),
        mesh=mesh,
        scratch_types=[
            pltpu.VMEM((_RPW,), jnp.int32),
            pltpu.VMEM((2, _C, _W), jnp.float32),
            pltpu.VMEM((2, _C, _W), jnp.float32),
            pltpu.VMEM((2, _C // 4, _W), jnp.int32),
            pltpu.VMEM((2, _C, _W), jnp.float32),
            pltpu.SemaphoreType.DMA((2,)),
            pltpu.SemaphoreType.DMA((2,)),
            pltpu.SemaphoreType.DMA((2,)),
        ],
    )
    return run(x, idx, m8)


def kernel(x, shuffled_idx, mask):
    return _cutmix_sc(x, shuffled_idx, mask.view(jnp.int8))
